# baseline (speedup 1.0000x reference)
import functools

import jax
import jax.numpy as jnp
from jax import lax
from jax.experimental import pallas as pl
from jax.experimental.pallas import tpu as pltpu
from jax.experimental.pallas import tpu_sc as plsc

NC = 2
NS = 16
L = 16
NW = NC * NS

Q = 2048
R = 32768
R_SC = 4096
R_TC = R - R_SC
QPW = Q // NW
QT = 4
NQT = QPW // QT
RV = R_SC // L
RU = 4
NRI = RV // RU

BR = 2048
NB = R_TC // BR
OFF_B = R_SC // BR

_mesh = plsc.VectorSubcoreMesh(
    core_axis_name="c", subcore_axis_name="s", num_cores=NC, num_subcores=NS
)


def _round_bf16(v):
  u = lax.bitcast_convert_type(v, jnp.uint32)
  odd = lax.shift_right_logical(u, jnp.uint32(16)) & jnp.uint32(1)
  r = (u + jnp.uint32(0x7FFF) + odd) & jnp.uint32(0xFFFF0000)
  return lax.bitcast_convert_type(r, jnp.float32)


@functools.partial(
    pl.kernel,
    out_type=jax.ShapeDtypeStruct((NW, QPW), jnp.float32),
    mesh=_mesh,
    compiler_params=pltpu.CompilerParams(needs_layout_passes=False),
    scratch_types=[
        pltpu.VMEM((QPW * 3,), jnp.float32),
        pltpu.VMEM((R_SC * 3,), jnp.float32),
        pltpu.VMEM((R_SC,), jnp.float32),
        pltpu.VMEM((R_SC,), jnp.float32),
        pltpu.VMEM((R_SC,), jnp.float32),
        pltpu.VMEM((R_SC,), jnp.float32),
        pltpu.VMEM((QPW * L,), jnp.float32),
        pltpu.VMEM((QPW,), jnp.float32),
    ],
)
def _chamfer_sc(q_hbm, r_hbm, out_hbm,
                q_v, raw_v, rx_v, ry_v, rz_v, rr_v, acc_v, sv):
  wid = lax.axis_index("c") * NS + lax.axis_index("s")

  pltpu.sync_copy(q_hbm.at[pl.ds(wid * QPW * 3, QPW * 3)], q_v)
  pltpu.sync_copy(r_hbm.at[pl.ds(0, R_SC * 3)], raw_v)

  lanes = lax.iota(jnp.int32, L)
  inf16 = jnp.full((L,), jnp.inf, dtype=jnp.float32)
  zero16 = jnp.zeros((L,), dtype=jnp.int32)

  def prologue(j, carry):
    off = j * L
    rows3 = (off + lanes) * 3
    x = plsc.load_gather(raw_v, [rows3])
    y = plsc.load_gather(raw_v, [rows3 + 1])
    z = plsc.load_gather(raw_v, [rows3 + 2])
    rr_v[pl.ds(off, L)] = x * x + y * y + z * z
    rx_v[pl.ds(off, L)] = _round_bf16(x)
    ry_v[pl.ds(off, L)] = _round_bf16(y)
    rz_v[pl.ds(off, L)] = _round_bf16(z)
    return carry

  lax.fori_loop(0, RV, prologue, jnp.int32(0))

  def qtile_body(qt, carry):
    qq = []
    ax = []
    ay = []
    az = []
    for t in range(QT):
      qrow3 = zero16 + (qt * QT + t) * 3
      qxv = plsc.load_gather(q_v, [qrow3])
      qyv = plsc.load_gather(q_v, [qrow3 + 1])
      qzv = plsc.load_gather(q_v, [qrow3 + 2])
      qq.append(qxv * qxv + qyv * qyv + qzv * qzv)
      ax.append(-2.0 * _round_bf16(qxv))
      ay.append(-2.0 * _round_bf16(qyv))
      az.append(-2.0 * _round_bf16(qzv))

    def rbody(i, accs):
      accs = list(accs)
      for u in range(RU):
        base = (i * RU + u) * L
        rxv = rx_v[pl.ds(base, L)]
        ryv = ry_v[pl.ds(base, L)]
        rzv = rz_v[pl.ds(base, L)]
        rrv = rr_v[pl.ds(base, L)]
        for t in range(QT):
          d = rrv + rxv * ax[t] + ryv * ay[t] + rzv * az[t]
          accs[t] = jnp.minimum(accs[t], d)
      return tuple(accs)

    accs = lax.fori_loop(0, NRI, rbody, (inf16,) * QT)
    for t in range(QT):
      soff = (qt * QT + t) * L
      acc_v[pl.ds(soff, L)] = accs[t] + qq[t]
    return carry

  lax.fori_loop(0, NQT, qtile_body, jnp.int32(0))

  for g in range(QPW // L):
    m = None
    for j in range(L):
      col = plsc.load_gather(acc_v, [lanes * L + (g * L * L + j)])
      m = col if m is None else jnp.minimum(m, col)
    sv[pl.ds(g * L, L)] = m
  pltpu.sync_copy(sv, out_hbm.at[wid])


def _tc_body(rf_ref, qf_ref, out_ref):
  j = pl.program_id(0)

  rf = rf_ref[...]
  qf = qf_ref[...]
  rr = jnp.sum(rf * rf, axis=1, keepdims=True)
  rb = rf.astype(jnp.bfloat16)
  qb2 = qf.astype(jnp.bfloat16) * jnp.bfloat16(-2.0)
  dots = lax.dot_general(rb, qb2, (((1,), (1,)), ((), ())),
                         preferred_element_type=jnp.float32)
  m = jnp.min(dots + rr, axis=0, keepdims=True)
  qq = jnp.sum(qf * qf, axis=1)[None, :]
  mq = m + qq

  @pl.when(j == 0)
  def _():
    out_ref[...] = mq

  @pl.when(j > 0)
  def _():
    out_ref[...] = jnp.minimum(out_ref[...], mq)


_chamfer_tc = pl.pallas_call(
    _tc_body,
    grid=(NB,),
    in_specs=[
        pl.BlockSpec((BR, 3), lambda j: (j + OFF_B, 0)),
        pl.BlockSpec((Q, 3), lambda j: (0, 0)),
    ],
    out_specs=pl.BlockSpec((1, Q), lambda j: (0, 0)),
    out_shape=jax.ShapeDtypeStruct((1, Q), jnp.float32),
    compiler_params=pltpu.CompilerParams(
        dimension_semantics=("arbitrary",),
    ),
)


def kernel(query, ref):
  sc_mins = _chamfer_sc(query.reshape(Q * 3), ref.reshape(R * 3))
  tc_mins = _chamfer_tc(ref, query)
  mins = jnp.minimum(sc_mins.reshape(Q), tc_mins.reshape(Q))
  return jnp.sum(mins) / jnp.float32(Q)


# speedup vs baseline: 1.1122x; 1.1122x over previous
import functools

import jax
import jax.numpy as jnp
from jax import lax
from jax.experimental import pallas as pl
from jax.experimental.pallas import tpu as pltpu
from jax.experimental.pallas import tpu_sc as plsc

NC = 2
NS = 16
L = 16
NW = NC * NS

Q = 2048
R = 32768
R_SC = 4096
R_TC = R - R_SC
QPW = Q // NW
QT = 4
NQT = QPW // QT
RV = R_SC // L
RU = 4
NRI = RV // RU

BR = 2048
NB = R_TC // BR
OFF_B = R_SC // BR

_mesh = plsc.VectorSubcoreMesh(
    core_axis_name="c", subcore_axis_name="s", num_cores=NC, num_subcores=NS
)


def _round_bf16(v):
  u = lax.bitcast_convert_type(v, jnp.uint32)
  odd = lax.shift_right_logical(u, jnp.uint32(16)) & jnp.uint32(1)
  r = (u + jnp.uint32(0x7FFF) + odd) & jnp.uint32(0xFFFF0000)
  return lax.bitcast_convert_type(r, jnp.float32)


@functools.partial(
    pl.kernel,
    out_type=jax.ShapeDtypeStruct((NW, QPW), jnp.float32),
    mesh=_mesh,
    compiler_params=pltpu.CompilerParams(needs_layout_passes=False),
    scratch_types=[
        pltpu.VMEM((QPW * 3,), jnp.float32),
        pltpu.VMEM((R_SC * 3,), jnp.float32),
        pltpu.VMEM((R_SC,), jnp.float32),
        pltpu.VMEM((R_SC,), jnp.float32),
        pltpu.VMEM((R_SC,), jnp.float32),
        pltpu.VMEM((R_SC,), jnp.float32),
        pltpu.VMEM((QPW * L,), jnp.float32),
        pltpu.VMEM((QPW,), jnp.float32),
    ],
)
def _chamfer_sc(q_hbm, r_hbm, out_hbm,
                q_v, raw_v, rx_v, ry_v, rz_v, rr_v, acc_v, sv):
  wid = lax.axis_index("c") * NS + lax.axis_index("s")

  for c in range(3):
    pltpu.sync_copy(q_hbm.at[pl.ds(c * Q + wid * QPW, QPW)],
                    q_v.at[pl.ds(c * QPW, QPW)])
  pltpu.sync_copy(r_hbm, raw_v)

  lanes = lax.iota(jnp.int32, L)
  inf16 = jnp.full((L,), jnp.inf, dtype=jnp.float32)
  zero16 = jnp.zeros((L,), dtype=jnp.int32)

  def prologue(j, carry):
    off = j * L
    rows3 = (off + lanes) * 3
    x = plsc.load_gather(raw_v, [rows3])
    y = plsc.load_gather(raw_v, [rows3 + 1])
    z = plsc.load_gather(raw_v, [rows3 + 2])
    rr_v[pl.ds(off, L)] = x * x + y * y + z * z
    rx_v[pl.ds(off, L)] = _round_bf16(x)
    ry_v[pl.ds(off, L)] = _round_bf16(y)
    rz_v[pl.ds(off, L)] = _round_bf16(z)
    return carry

  lax.fori_loop(0, RV, prologue, jnp.int32(0))

  def qtile_body(qt, carry):
    qq = []
    ax = []
    ay = []
    az = []
    for t in range(QT):
      qrow = zero16 + (qt * QT + t)
      qxv = plsc.load_gather(q_v, [qrow])
      qyv = plsc.load_gather(q_v, [qrow + QPW])
      qzv = plsc.load_gather(q_v, [qrow + 2 * QPW])
      qq.append(qxv * qxv + qyv * qyv + qzv * qzv)
      ax.append(-2.0 * _round_bf16(qxv))
      ay.append(-2.0 * _round_bf16(qyv))
      az.append(-2.0 * _round_bf16(qzv))

    def rbody(i, accs):
      accs = list(accs)
      for u in range(RU):
        base = (i * RU + u) * L
        rxv = rx_v[pl.ds(base, L)]
        ryv = ry_v[pl.ds(base, L)]
        rzv = rz_v[pl.ds(base, L)]
        rrv = rr_v[pl.ds(base, L)]
        for t in range(QT):
          d = rrv + rxv * ax[t] + ryv * ay[t] + rzv * az[t]
          accs[t] = jnp.minimum(accs[t], d)
      return tuple(accs)

    accs = lax.fori_loop(0, NRI, rbody, (inf16,) * QT)
    for t in range(QT):
      soff = (qt * QT + t) * L
      acc_v[pl.ds(soff, L)] = accs[t] + qq[t]
    return carry

  lax.fori_loop(0, NQT, qtile_body, jnp.int32(0))

  for g in range(QPW // L):
    m = None
    for j in range(L):
      col = plsc.load_gather(acc_v, [lanes * L + (g * L * L + j)])
      m = col if m is None else jnp.minimum(m, col)
    sv[pl.ds(g * L, L)] = m
  pltpu.sync_copy(sv, out_hbm.at[wid])


def _tc_body(rf_ref, qf_ref, out_ref):
  j = pl.program_id(0)

  rf = rf_ref[...]
  qf = qf_ref[...]
  rr = jnp.sum(rf * rf, axis=1, keepdims=True)
  rb = rf.astype(jnp.bfloat16)
  qb2 = qf.astype(jnp.bfloat16) * jnp.bfloat16(-2.0)
  dots = lax.dot_general(rb, qb2, (((1,), (1,)), ((), ())),
                         preferred_element_type=jnp.float32)
  m = jnp.min(dots + rr, axis=0, keepdims=True)
  qq = jnp.sum(qf * qf, axis=1)[None, :]
  mq = m + qq

  @pl.when(j == 0)
  def _():
    out_ref[...] = mq

  @pl.when(j > 0)
  def _():
    out_ref[...] = jnp.minimum(out_ref[...], mq)


_chamfer_tc = pl.pallas_call(
    _tc_body,
    grid=(NB,),
    in_specs=[
        pl.BlockSpec((BR, 3), lambda j: (j + OFF_B, 0)),
        pl.BlockSpec((Q, 3), lambda j: (0, 0)),
    ],
    out_specs=pl.BlockSpec((1, Q), lambda j: (0, 0)),
    out_shape=jax.ShapeDtypeStruct((1, Q), jnp.float32),
    compiler_params=pltpu.CompilerParams(
        dimension_semantics=("arbitrary",),
    ),
)


def kernel(query, ref):
  qt_flat = query.T.reshape(3 * Q)
  ref_sc = ref[:R_SC].reshape(R_SC * 3)
  sc_mins = _chamfer_sc(qt_flat, ref_sc)
  tc_mins = _chamfer_tc(ref, query)
  mins = jnp.minimum(sc_mins.reshape(Q), tc_mins.reshape(Q))
  return jnp.sum(mins) / jnp.float32(Q)
